# baseline (speedup 1.0000x reference)
import functools

import jax
import jax.numpy as jnp
from jax import lax
from jax.experimental import pallas as pl
from jax.experimental.pallas import tpu as pltpu
from jax.experimental.pallas import tpu_sc as plsc

EMBED_DIM = 64
PAD_DIM = 128
NUM_CORES = 2
NUM_SUBCORES = 16
NUM_WORKERS = NUM_CORES * NUM_SUBCORES
CHUNK = 160
IDXW = 256
NBUF = 2


def kernel(x, weight):
    batch, hist = x.shape
    num_indices = batch * hist
    idx = x.reshape(num_indices).astype(jnp.int32)
    per_worker = num_indices // NUM_WORKERS
    rows_per_worker = batch // NUM_WORKERS
    rows_per_chunk = CHUNK // hist
    n_chunks = per_worker // CHUNK
    w128 = jnp.pad(weight, ((0, 0), (0, PAD_DIM - EMBED_DIM)))

    mesh = plsc.VectorSubcoreMesh(core_axis_name="c", subcore_axis_name="s")

    row_buf = pltpu.VMEM((CHUNK, PAD_DIM), jnp.float32)

    @functools.partial(
        pl.kernel,
        mesh=mesh,
        out_type=jax.ShapeDtypeStruct((batch, hist, PAD_DIM), weight.dtype),
        scratch_types=[
            pltpu.VMEM((per_worker,), jnp.int32),
            *([row_buf] * NBUF),
            *([pltpu.SemaphoreType.DMA] * (2 * NBUF)),
        ],
        compiler_params=pltpu.CompilerParams(use_tc_tiling_on_sc=True),
    )
    def gather_kernel(idx_hbm, table_hbm, out_hbm, idx_v, *bufs_sems):
        rbufs = bufs_sems[:NBUF]
        gsems = bufs_sems[NBUF:2 * NBUF]
        wsems = bufs_sems[2 * NBUF:]

        wid = lax.axis_index("s") * NUM_CORES + lax.axis_index("c")
        base = wid * per_worker
        row_base = wid * rows_per_worker

        pltpu.sync_copy(idx_hbm.at[pl.ds(base, per_worker)], idx_v)

        def idx_slice(c):
            return idx_v.at[pl.ds(c * CHUNK, CHUNK)]

        def wb_start(b, c):
            for r in range(rows_per_chunk):
                pltpu.async_copy(
                    rbufs[b].at[pl.ds(r * hist, hist), :],
                    out_hbm.at[row_base + c * rows_per_chunk + r],
                    wsems[b])

        def wb_wait(b, c):
            for r in range(rows_per_chunk):
                pltpu.make_async_copy(
                    rbufs[b].at[pl.ds(r * hist, hist), :],
                    out_hbm.at[row_base + c * rows_per_chunk + r],
                    wsems[b]).wait()

        @pl.loop(0, n_chunks)
        def _(c):
            pltpu.async_copy(table_hbm.at[idx_slice(c)], rbufs[0], gsems[0])
            pltpu.make_async_copy(table_hbm.at[idx_slice(c)],
                                  rbufs[0], gsems[0]).wait()
            wb_start(0, c)
            wb_wait(0, c)

    return gather_kernel(idx, w128)[:, :, :EMBED_DIM]


# speedup vs baseline: 1.0349x; 1.0349x over previous
import functools

import jax
import jax.numpy as jnp
from jax import lax
from jax.experimental import pallas as pl
from jax.experimental.pallas import tpu as pltpu
from jax.experimental.pallas import tpu_sc as plsc

EMBED_DIM = 64
PAD_DIM = 128
NUM_CORES = 2
NUM_SUBCORES = 16
NUM_WORKERS = NUM_CORES * NUM_SUBCORES
CHUNK = 160
IDXW = 256
NBUF = 2


def kernel(x, weight):
    batch, hist = x.shape
    num_indices = batch * hist
    idx = x.reshape(num_indices).astype(jnp.int32)
    per_worker = num_indices // NUM_WORKERS
    rows_per_worker = batch // NUM_WORKERS
    rows_per_chunk = CHUNK // hist
    n_chunks = per_worker // CHUNK
    w128 = jnp.pad(weight, ((0, 0), (0, PAD_DIM - EMBED_DIM)))

    mesh = plsc.VectorSubcoreMesh(core_axis_name="c", subcore_axis_name="s")

    row_buf = pltpu.VMEM((CHUNK, PAD_DIM), jnp.float32)

    @functools.partial(
        pl.kernel,
        mesh=mesh,
        out_type=jax.ShapeDtypeStruct((batch, hist, PAD_DIM), weight.dtype),
        scratch_types=[
            pltpu.VMEM((per_worker,), jnp.int32),
            *([row_buf] * NBUF),
            *([pltpu.SemaphoreType.DMA] * (2 * NBUF)),
        ],
        compiler_params=pltpu.CompilerParams(use_tc_tiling_on_sc=True),
    )
    def gather_kernel(idx_hbm, table_hbm, out_hbm, idx_v, *bufs_sems):
        rbufs = bufs_sems[:NBUF]
        gsems = bufs_sems[NBUF:2 * NBUF]
        wsems = bufs_sems[2 * NBUF:]

        wid = lax.axis_index("s") * NUM_CORES + lax.axis_index("c")
        base = wid * per_worker
        row_base = wid * rows_per_worker

        pltpu.sync_copy(idx_hbm.at[pl.ds(base, per_worker)], idx_v)

        def idx_slice(c):
            return idx_v.at[pl.ds(c * CHUNK, CHUNK)]

        def wb_start(b, c):
            for r in range(rows_per_chunk):
                pltpu.async_copy(
                    rbufs[b].at[pl.ds(r * hist, hist), :],
                    out_hbm.at[row_base + c * rows_per_chunk + r],
                    wsems[b])

        def wb_wait(b, c):
            for r in range(rows_per_chunk):
                pltpu.make_async_copy(
                    rbufs[b].at[pl.ds(r * hist, hist), :],
                    out_hbm.at[row_base + c * rows_per_chunk + r],
                    wsems[b]).wait()

        pltpu.async_copy(table_hbm.at[idx_slice(0)], rbufs[0], gsems[0])

        @pl.loop(0, n_chunks, step=NBUF)
        def _(k):
            for b in range(NBUF):
                c = k + b
                nb = 1 - b
                pltpu.make_async_copy(table_hbm.at[idx_slice(c)],
                                      rbufs[b], gsems[b]).wait()
                wb_start(b, c)

                @pl.when(c + 1 < n_chunks)
                def _():
                    @pl.when(c >= 1)
                    def _():
                        wb_wait(nb, c - 1)

                    pltpu.async_copy(table_hbm.at[idx_slice(c + 1)],
                                     rbufs[nb], gsems[nb])

        wb_wait(0, n_chunks - 2)
        wb_wait(1, n_chunks - 1)

    return gather_kernel(idx, w128)[:, :, :EMBED_DIM]


# speedup vs baseline: 1.0603x; 1.0246x over previous
import functools

import jax
import jax.numpy as jnp
from jax import lax
from jax.experimental import pallas as pl
from jax.experimental.pallas import tpu as pltpu
from jax.experimental.pallas import tpu_sc as plsc

EMBED_DIM = 64
PAD_DIM = 128
NUM_CORES = 2
NUM_SUBCORES = 16
NUM_WORKERS = NUM_CORES * NUM_SUBCORES
CHUNK = 320
IDXW = 256
NBUF = 2


def kernel(x, weight):
    batch, hist = x.shape
    num_indices = batch * hist
    idx = x.reshape(num_indices).astype(jnp.int32)
    per_worker = num_indices // NUM_WORKERS
    rows_per_worker = batch // NUM_WORKERS
    rows_per_chunk = CHUNK // hist
    n_chunks = per_worker // CHUNK
    w128 = jnp.pad(weight, ((0, 0), (0, PAD_DIM - EMBED_DIM)))

    mesh = plsc.VectorSubcoreMesh(core_axis_name="c", subcore_axis_name="s")

    row_buf = pltpu.VMEM((CHUNK, PAD_DIM), jnp.float32)

    @functools.partial(
        pl.kernel,
        mesh=mesh,
        out_type=jax.ShapeDtypeStruct((batch, hist, PAD_DIM), weight.dtype),
        scratch_types=[
            pltpu.VMEM((per_worker,), jnp.int32),
            *([row_buf] * NBUF),
            *([pltpu.SemaphoreType.DMA] * (2 * NBUF)),
        ],
        compiler_params=pltpu.CompilerParams(use_tc_tiling_on_sc=True),
    )
    def gather_kernel(idx_hbm, table_hbm, out_hbm, idx_v, *bufs_sems):
        rbufs = bufs_sems[:NBUF]
        gsems = bufs_sems[NBUF:2 * NBUF]
        wsems = bufs_sems[2 * NBUF:]

        wid = lax.axis_index("s") * NUM_CORES + lax.axis_index("c")
        base = wid * per_worker
        row_base = wid * rows_per_worker

        pltpu.sync_copy(idx_hbm.at[pl.ds(base, per_worker)], idx_v)

        def idx_slice(c):
            return idx_v.at[pl.ds(c * CHUNK, CHUNK)]

        def wb_start(b, c):
            for r in range(rows_per_chunk):
                pltpu.async_copy(
                    rbufs[b].at[pl.ds(r * hist, hist), :],
                    out_hbm.at[row_base + c * rows_per_chunk + r],
                    wsems[b])

        def wb_wait(b, c):
            for r in range(rows_per_chunk):
                pltpu.make_async_copy(
                    rbufs[b].at[pl.ds(r * hist, hist), :],
                    out_hbm.at[row_base + c * rows_per_chunk + r],
                    wsems[b]).wait()

        pltpu.async_copy(table_hbm.at[idx_slice(0)], rbufs[0], gsems[0])

        @pl.loop(0, n_chunks, step=NBUF)
        def _(k):
            for b in range(NBUF):
                c = k + b
                nb = 1 - b
                pltpu.make_async_copy(table_hbm.at[idx_slice(c)],
                                      rbufs[b], gsems[b]).wait()
                wb_start(b, c)

                @pl.when(c + 1 < n_chunks)
                def _():
                    @pl.when(c >= 1)
                    def _():
                        wb_wait(nb, c - 1)

                    pltpu.async_copy(table_hbm.at[idx_slice(c + 1)],
                                     rbufs[nb], gsems[nb])

        wb_wait(0, n_chunks - 2)
        wb_wait(1, n_chunks - 1)

    return gather_kernel(idx, w128)[:, :, :EMBED_DIM]


# speedup vs baseline: 1.0622x; 1.0018x over previous
import functools

import jax
import jax.numpy as jnp
from jax import lax
from jax.experimental import pallas as pl
from jax.experimental.pallas import tpu as pltpu
from jax.experimental.pallas import tpu_sc as plsc

EMBED_DIM = 64
PAD_DIM = 128
NUM_CORES = 2
NUM_SUBCORES = 16
NUM_WORKERS = NUM_CORES * NUM_SUBCORES
CHUNK = 320
IDXW = 256
NBUF = 2


def kernel(x, weight):
    batch, hist = x.shape
    num_indices = batch * hist
    per_worker = num_indices // NUM_WORKERS
    rows_per_worker = batch // NUM_WORKERS
    rows_per_chunk = CHUNK // hist
    n_chunks = per_worker // CHUNK
    w128 = jnp.pad(weight, ((0, 0), (0, PAD_DIM - EMBED_DIM)))
    idx = x.reshape(num_indices).astype(jnp.int32)

    mesh = plsc.VectorSubcoreMesh(core_axis_name="c", subcore_axis_name="s")

    row_buf = pltpu.VMEM((CHUNK, PAD_DIM), jnp.float32)

    @functools.partial(
        pl.kernel,
        mesh=mesh,
        out_type=jax.ShapeDtypeStruct((batch, hist, PAD_DIM), weight.dtype),
        scratch_types=[
            pltpu.VMEM((per_worker,), jnp.int32),
            *([row_buf] * NBUF),
            *([pltpu.SemaphoreType.DMA] * (2 * NBUF)),
        ],
        compiler_params=pltpu.CompilerParams(use_tc_tiling_on_sc=True),
    )
    def gather_kernel(idx_hbm, table_hbm, out_hbm, idx_v, *bufs_sems):
        rbufs = bufs_sems[:NBUF]
        gsems = bufs_sems[NBUF:2 * NBUF]
        wsems = bufs_sems[2 * NBUF:]

        wid = lax.axis_index("s") * NUM_CORES + lax.axis_index("c")
        base = wid * per_worker
        row_base = wid * rows_per_worker

        pltpu.sync_copy(idx_hbm.at[pl.ds(base, per_worker)], idx_v)

        def idx_slice(c):
            return idx_v.at[pl.ds(c * CHUNK, CHUNK)]

        def wb_start(b, c):
            for r in range(rows_per_chunk):
                pltpu.async_copy(
                    rbufs[b].at[pl.ds(r * hist, hist), :],
                    out_hbm.at[row_base + c * rows_per_chunk + r],
                    wsems[b])

        def wb_wait(b, c):
            for r in range(rows_per_chunk):
                pltpu.make_async_copy(
                    rbufs[b].at[pl.ds(r * hist, hist), :],
                    out_hbm.at[row_base + c * rows_per_chunk + r],
                    wsems[b]).wait()

        pltpu.async_copy(table_hbm.at[idx_slice(0)], rbufs[0], gsems[0])

        @pl.loop(0, n_chunks, step=NBUF)
        def _(k):
            for b in range(NBUF):
                c = k + b
                nb = 1 - b
                pltpu.make_async_copy(table_hbm.at[idx_slice(c)],
                                      rbufs[b], gsems[b]).wait()
                wb_start(b, c)

                @pl.when(c + 1 < n_chunks)
                def _():
                    @pl.when(c >= 1)
                    def _():
                        wb_wait(nb, c - 1)

                    pltpu.async_copy(table_hbm.at[idx_slice(c + 1)],
                                     rbufs[nb], gsems[nb])

        wb_wait(0, n_chunks - 2)
        wb_wait(1, n_chunks - 1)

    return gather_kernel(idx, w128)[:, :, :EMBED_DIM]
